# Initial kernel scaffold; baseline (speedup 1.0000x reference)
import jax
import jax.numpy as jnp
from jax.experimental import pallas as pl


def kernel(node_embeds, edge_index, Wq, bq, Wk, bk, Wv, bv, gamma, beta):
    raise NotImplementedError("write your pallas kernel here")


# speedup vs baseline: 1.7915x; 1.7915x over previous
import functools

import jax
import jax.numpy as jnp
from jax import lax
from jax.experimental import pallas as pl
from jax.experimental.pallas import tpu as pltpu
from jax.experimental.pallas import tpu_sc as plsc

D_MODEL = 128
N_HEADS = 8
HEAD_DIM = 16
N_NODES = 10000
N_EDGES = 320000

NC = 2
NS = 16
NW = NC * NS
EDGES_PER_WORKER = N_EDGES // NW
BLK = 40
N_BLOCKS = EDGES_PER_WORKER // BLK
ZROWS = 48
ROWS_PER_TILE = 624


def _qkv_body(x_ref, wqt_ref, bq_ref, wkt_ref, bk_ref, wvt_ref, bv_ref,
              q_ref, k_ref, v_ref):
    x = x_ref[...]
    q_ref[...] = jnp.dot(x, wqt_ref[...],
                         preferred_element_type=jnp.float32) + bq_ref[...]
    k_ref[...] = (jnp.dot(x, wkt_ref[...],
                          preferred_element_type=jnp.float32) + bk_ref[...]) * 0.25
    v_ref[...] = jnp.dot(x, wvt_ref[...],
                         preferred_element_type=jnp.float32) + bv_ref[...]


_GATHER_DNUMS = lax.GatherDimensionNumbers(
    offset_dims=(), collapsed_slice_dims=(0,), start_index_map=(0,))


def _lane_shuffle(x, idx):
    return lax.gather(x, idx[:, None], _GATHER_DNUMS, (1,),
                      mode=lax.GatherScatterMode.PROMISE_IN_BOUNDS)


def _edge_body(q_hbm, k_hbm, v_hbm, rows_hbm, cols_hbm,
               wv_out, att_out,
               rows_v, cols_v, q_v, k_v, v_v, wv_v, att_v, zrow_v, zatt_v,
               acc_sh, att_sh, semq, semk, semv):
    cid = lax.axis_index("c")
    sid = lax.axis_index("s")
    wid = cid * NS + sid

    for j in range(D_MODEL // 16):
        for i in range(ZROWS):
            zrow_v[i, pl.ds(j * 16, 16)] = jnp.zeros((16,), jnp.float32)
    for i in range(ZROWS):
        zatt_v[i, pl.ds(0, 16)] = jnp.zeros((16,), jnp.float32)
    base_row = sid * ROWS_PER_TILE
    for r in range(ROWS_PER_TILE // ZROWS):
        pltpu.sync_copy(zrow_v, acc_sh.at[pl.ds(base_row + r * ZROWS, ZROWS)])
        pltpu.sync_copy(zatt_v, att_sh.at[pl.ds(base_row + r * ZROWS, ZROWS)])
    tail_row = NS * ROWS_PER_TILE

    @pl.when(sid == 0)
    def _zero_tail():
        pltpu.sync_copy(zrow_v.at[pl.ds(0, 16)], acc_sh.at[pl.ds(tail_row, 16)])
        pltpu.sync_copy(zatt_v.at[pl.ds(0, 16)], att_sh.at[pl.ds(tail_row, 16)])

    plsc.subcore_barrier()

    lane = lax.iota(jnp.int32, 16)
    edge_base = wid * EDGES_PER_WORKER

    def block_body(blk, carry):
        b0 = edge_base + blk * BLK
        pltpu.sync_copy(rows_hbm.at[pl.ds(b0, BLK)], rows_v)
        pltpu.sync_copy(cols_hbm.at[pl.ds(b0, BLK)], cols_v)
        cq = pltpu.async_copy(q_hbm.at[rows_v], q_v, semq)
        ck = pltpu.async_copy(k_hbm.at[cols_v], k_v, semk)
        cv = pltpu.async_copy(v_hbm.at[cols_v], v_v, semv)
        cq.wait()
        ck.wait()
        cv.wait()

        def edge_body(e, carry2):
            merged = jnp.zeros((16,), jnp.float32)
            for h in range(N_HEADS):
                qh = q_v[e, pl.ds(h * 16, 16)]
                kh = k_v[e, pl.ds(h * 16, 16)]
                att = qh * kh
                for step in (8, 4, 2, 1):
                    att = att + _lane_shuffle(att, lane ^ step)
                att = jnp.minimum(jnp.maximum(att, -10.0), 10.0)
                ex = jnp.exp(att)
                wv_v[e, pl.ds(h * 16, 16)] = ex * v_v[e, pl.ds(h * 16, 16)]
                merged = jnp.where(lane == h, ex, merged)
            att_v[e, pl.ds(0, 16)] = merged
            return carry2

        lax.fori_loop(0, BLK, edge_body, 0, unroll=2)
        pltpu.sync_copy(wv_v, acc_sh.at[rows_v], add=True)
        pltpu.sync_copy(att_v, att_sh.at[rows_v], add=True)
        return carry

    lax.fori_loop(0, N_BLOCKS, block_body, 0)
    plsc.subcore_barrier()

    pltpu.sync_copy(acc_sh.at[pl.ds(base_row, ROWS_PER_TILE)],
                    wv_out.at[cid, pl.ds(base_row, ROWS_PER_TILE)])
    pltpu.sync_copy(att_sh.at[pl.ds(base_row, ROWS_PER_TILE)],
                    att_out.at[cid, pl.ds(base_row, ROWS_PER_TILE)])

    @pl.when(sid == 0)
    def _copy_tail():
        pltpu.sync_copy(acc_sh.at[pl.ds(tail_row, 16)],
                        wv_out.at[cid, pl.ds(tail_row, 16)])
        pltpu.sync_copy(att_sh.at[pl.ds(tail_row, 16)],
                        att_out.at[cid, pl.ds(tail_row, 16)])


@functools.partial(jax.jit, static_argnames=())
def _edge_pass(q, k, v, rows, cols):
    mesh = plsc.VectorSubcoreMesh(core_axis_name="c", subcore_axis_name="s")
    fn = pl.kernel(
        _edge_body,
        out_type=[
            jax.ShapeDtypeStruct((NC, N_NODES, D_MODEL), jnp.float32),
            jax.ShapeDtypeStruct((NC, N_NODES, 16), jnp.float32),
        ],
        mesh=mesh,
        scratch_types=[
            pltpu.VMEM((BLK,), jnp.int32),
            pltpu.VMEM((BLK,), jnp.int32),
            pltpu.VMEM((BLK, D_MODEL), jnp.float32),
            pltpu.VMEM((BLK, D_MODEL), jnp.float32),
            pltpu.VMEM((BLK, D_MODEL), jnp.float32),
            pltpu.VMEM((BLK, D_MODEL), jnp.float32),
            pltpu.VMEM((BLK, 16), jnp.float32),
            pltpu.VMEM((ZROWS, D_MODEL), jnp.float32),
            pltpu.VMEM((ZROWS, 16), jnp.float32),
            pltpu.VMEM_SHARED((N_NODES, D_MODEL), jnp.float32),
            pltpu.VMEM_SHARED((N_NODES, 16), jnp.float32),
            pltpu.SemaphoreType.DMA,
            pltpu.SemaphoreType.DMA,
            pltpu.SemaphoreType.DMA,
        ],
        compiler_params=pltpu.CompilerParams(use_tc_tiling_on_sc=False),
    )
    return fn(q, k, v, rows, cols)


def _final_body(acc_ref, att_ref, x_ref, gamma_ref, beta_ref, out_ref):
    unnorm = acc_ref[0] + acc_ref[1]
    s16 = att_ref[0] + att_ref[1]
    col = lax.broadcasted_iota(jnp.int32, (16, D_MODEL), 1) // HEAD_DIM
    row = lax.broadcasted_iota(jnp.int32, (16, D_MODEL), 0)
    onehot = jnp.where(col == row, 1.0, 0.0).astype(jnp.float32)
    denom = jnp.dot(s16, onehot, preferred_element_type=jnp.float32) + 1e-8
    out = unnorm / denom + x_ref[...]
    mean = jnp.mean(out, axis=-1, keepdims=True)
    var = jnp.mean((out - mean) ** 2, axis=-1, keepdims=True)
    out_ref[...] = ((out - mean) / jnp.sqrt(var + 1e-6)) * gamma_ref[...] \
        + beta_ref[...]


def kernel(node_embeds, edge_index, Wq, bq, Wk, bk, Wv, bv, gamma, beta):
    rows = edge_index[0].astype(jnp.int32)
    cols = edge_index[1].astype(jnp.int32)

    q, k, v = pl.pallas_call(
        _qkv_body,
        out_shape=[jax.ShapeDtypeStruct((N_NODES, D_MODEL), jnp.float32)] * 3,
    )(node_embeds, Wq.T, bq, Wk.T, bk, Wv.T, bv)

    acc, att = _edge_pass(q, k, v, rows, cols)

    nrows = 400
    grid = N_NODES // nrows
    out = pl.pallas_call(
        _final_body,
        grid=(grid,),
        in_specs=[
            pl.BlockSpec((NC, nrows, D_MODEL), lambda i: (0, i, 0)),
            pl.BlockSpec((NC, nrows, 16), lambda i: (0, i, 0)),
            pl.BlockSpec((nrows, D_MODEL), lambda i: (i, 0)),
            pl.BlockSpec((D_MODEL,), lambda i: (0,)),
            pl.BlockSpec((D_MODEL,), lambda i: (0,)),
        ],
        out_specs=pl.BlockSpec((nrows, D_MODEL), lambda i: (i, 0)),
        out_shape=jax.ShapeDtypeStruct((N_NODES, D_MODEL), jnp.float32),
    )(acc, att, node_embeds, gamma, beta)
    return out
